# baseline (speedup 1.0000x reference)
import jax
import jax.numpy as jnp
from jax import lax
from jax.experimental import pallas as pl
from jax.experimental.pallas import tpu as pltpu
from jax.experimental.pallas import tpu_sc as plsc


def _loss_body(table_hbm, ind_hbm, mask_hbm, tgt_hbm, out_hbm,
               ind_v, mask_v, tgt_v, plane_v, part_v, out_v,
               slab, red_v, sem, psem):
    B, M = 16, 128
    D, HW = 3, 128 * 128
    L = 16
    NCH = M // L
    beta = 1.0 / 9.0

    s = lax.axis_index("s")
    b = s

    rows = table_hbm.reshape(B * D * 128, 128)
    d_pln = pltpu.async_copy(
        rows.at[pl.ds(b * D * 128, D * 128)], plane_v, psem)
    d_ind = pltpu.async_copy(ind_hbm.at[b], ind_v, sem)
    d_msk = pltpu.async_copy(mask_hbm.at[b], mask_v, sem)
    d_tgt = pltpu.async_copy(tgt_hbm.at[pl.ds(b * M, M)], tgt_v, sem)
    d_ind.wait()
    d_msk.wait()
    d_tgt.wait()
    d_pln.wait()

    def chunk(c, carry):
        a0, a1, a2, ms = carry
        mk = mask_v[pl.ds(c * L, L)].astype(jnp.float32)
        ms = ms + mk
        ik = ind_v[pl.ds(c * L, L)]
        hi = lax.shift_right_logical(ik, 7)
        lo = lax.bitwise_and(ik, 127)
        midx = lax.iota(jnp.int32, L) + c * L
        accs = [a0, a1, a2]
        for d in range(D):
            pr = plsc.load_gather(plane_v, [hi + d * 128, lo])
            tg = plsc.load_gather(tgt_v, [midx, jnp.full((L,), d, jnp.int32)])
            m = mk * jnp.where(tg == tg, 1.0, 0.0).astype(jnp.float32)
            n = jnp.abs(pr * m - tg * m)
            loss = jnp.where(n < beta, 0.5 * n * n / beta, n - 0.5 * beta)
            accs[d] = accs[d] + loss
        return accs[0], accs[1], accs[2], ms

    z = jnp.zeros((L,), jnp.float32)
    acc0, acc1, acc2, msum = lax.fori_loop(0, NCH, chunk, (z, z, z, z))

    lane = lax.iota(jnp.int32, L)
    l0 = jnp.sum(acc0)
    l1 = jnp.sum(acc1)
    l2 = jnp.sum(acc2)
    ms = jnp.sum(msum)
    packed = jnp.where(
        lane == 0, l0,
        jnp.where(lane == 1, l1, jnp.where(lane == 2, l2, ms)))
    part_v[:] = packed

    pltpu.sync_copy(part_v, slab.at[s])
    plsc.subcore_barrier()

    @pl.when(s == 0)
    def _():
        pltpu.sync_copy(slab, red_v)
        tot = jnp.zeros((L,), jnp.float32)
        for w in range(16):
            tot = tot + red_v[w, :]
        num = jnp.sum(jnp.where(lane == 3, tot, 0.0))
        denom = jnp.full((L,), num + 0.0001, jnp.float32)
        out_v[:] = tot / denom
        pltpu.sync_copy(out_v.at[pl.ds(0, 3)], out_hbm)


@jax.jit
def _run(table, ind, mask, target):
    B, M = ind.shape
    D = 3
    mesh = plsc.VectorSubcoreMesh(
        core_axis_name="c", subcore_axis_name="s", num_cores=1
    )
    f = pl.kernel(
        _loss_body,
        out_type=jax.ShapeDtypeStruct((D,), jnp.float32),
        mesh=mesh,
        compiler_params=pltpu.CompilerParams(needs_layout_passes=False),
        scratch_types=[
            pltpu.VMEM((M,), jnp.int32),
            pltpu.VMEM((M,), jnp.int32),
            pltpu.VMEM((M, D), jnp.float32),
            pltpu.VMEM((D * 128, 128), jnp.float32),
            pltpu.VMEM((16,), jnp.float32),
            pltpu.VMEM((16,), jnp.float32),
            pltpu.HBM((16, 16), jnp.float32),
            pltpu.VMEM((16, 16), jnp.float32),
            pltpu.SemaphoreType.DMA,
            pltpu.SemaphoreType.DMA,
        ],
    )
    return f(table, ind, mask, target)


def kernel(output, mask, ind, target):
    B, D, H, W = output.shape
    M = ind.shape[1]
    return _run(output, ind, mask, target.reshape(B * M, D))


# speedup vs baseline: 1.0799x; 1.0799x over previous
import jax
import jax.numpy as jnp
from jax import lax
from jax.experimental import pallas as pl
from jax.experimental.pallas import tpu as pltpu
from jax.experimental.pallas import tpu_sc as plsc


def _loss_body(table_hbm, ind_hbm, mask_hbm, tgt_hbm, out_hbm,
               ind_v, mask_v, tgt_v, idx_v, pred_v, part_v, out_v,
               slab, red_v, sem, gsem):
    B, M = 16, 128
    D, HW = 3, 128 * 128
    L = 16
    NCH = M // L
    beta = 1.0 / 9.0

    s = lax.axis_index("s")
    b = s

    d_ind = pltpu.async_copy(ind_hbm.at[b], ind_v, sem)
    d_msk = pltpu.async_copy(mask_hbm.at[b], mask_v, sem)
    d_tgt = pltpu.async_copy(tgt_hbm.at[pl.ds(b * M, M)], tgt_v, sem)
    d_ind.wait()

    descs = []
    for d in range(D):
        base = (b * D + d) * HW
        for c in range(NCH):
            idx_v[d, pl.ds(c * L, L)] = ind_v[pl.ds(c * L, L)] + base
        descs.append(
            pltpu.async_copy(table_hbm.at[idx_v.at[d]], pred_v.at[d], gsem[d]))
    d_msk.wait()
    d_tgt.wait()

    def make_chunk(d, with_mask):
        def chunk(c, carry):
            a, ms = carry
            mk = mask_v[pl.ds(c * L, L)].astype(jnp.float32)
            if with_mask:
                ms = ms + mk
            midx = lax.iota(jnp.int32, L) + c * L
            pr = pred_v[d, pl.ds(c * L, L)]
            tg = plsc.load_gather(tgt_v, [midx, jnp.full((L,), d, jnp.int32)])
            m = mk * jnp.where(tg == tg, 1.0, 0.0).astype(jnp.float32)
            n = jnp.abs(pr * m - tg * m)
            loss = jnp.where(n < beta, 0.5 * n * n / beta, n - 0.5 * beta)
            return a + loss, ms
        return chunk

    z = jnp.zeros((L,), jnp.float32)
    acc = []
    msum = z
    for d in range(D):
        descs[d].wait()
        a, msum = lax.fori_loop(0, NCH, make_chunk(d, d == 0), (z, msum))
        acc.append(a)

    lane = lax.iota(jnp.int32, L)
    l0 = jnp.sum(acc[0])
    l1 = jnp.sum(acc[1])
    l2 = jnp.sum(acc[2])
    ms = jnp.sum(msum)
    packed = jnp.where(
        lane == 0, l0,
        jnp.where(lane == 1, l1, jnp.where(lane == 2, l2, ms)))
    part_v[:] = packed

    pltpu.sync_copy(part_v, slab.at[s])
    plsc.subcore_barrier()

    @pl.when(s == 0)
    def _():
        pltpu.sync_copy(slab, red_v)
        tot = jnp.zeros((L,), jnp.float32)
        for w in range(16):
            tot = tot + red_v[w, :]
        num = jnp.sum(jnp.where(lane == 3, tot, 0.0))
        denom = jnp.full((L,), num + 0.0001, jnp.float32)
        out_v[:] = tot / denom
        pltpu.sync_copy(out_v.at[pl.ds(0, 3)], out_hbm)


@jax.jit
def _run(table, ind, mask, target):
    B, M = ind.shape
    D = 3
    mesh = plsc.VectorSubcoreMesh(
        core_axis_name="c", subcore_axis_name="s", num_cores=1
    )
    f = pl.kernel(
        _loss_body,
        out_type=jax.ShapeDtypeStruct((D,), jnp.float32),
        mesh=mesh,
        compiler_params=pltpu.CompilerParams(needs_layout_passes=False),
        scratch_types=[
            pltpu.VMEM((M,), jnp.int32),
            pltpu.VMEM((M,), jnp.int32),
            pltpu.VMEM((M, D), jnp.float32),
            pltpu.VMEM((D, M), jnp.int32),
            pltpu.VMEM((D, M), jnp.float32),
            pltpu.VMEM((16,), jnp.float32),
            pltpu.VMEM((16,), jnp.float32),
            pltpu.HBM((16, 16), jnp.float32),
            pltpu.VMEM((16, 16), jnp.float32),
            pltpu.SemaphoreType.DMA,
            [pltpu.SemaphoreType.DMA] * 3,
        ],
    )
    return f(table, ind, mask, target)


def kernel(output, mask, ind, target):
    B, D, H, W = output.shape
    M = ind.shape[1]
    table = output.reshape(B * D * H * W)
    return _run(table, ind, mask, target.reshape(B * M, D))
